# Initial kernel scaffold; baseline (speedup 1.0000x reference)
import functools

import jax
import jax.numpy as jnp
from jax import lax
from jax.experimental import pallas as pl
from jax.experimental.pallas import tpu as pltpu
from jax.experimental.pallas import tpu_sc as plsc

_B = 16
_N = 50000
_E = 1600000
_NC = 2
_NS = 16
_NW = _NC * _NS
_EHALF = _E // _NC
_CHUNK = 4000
_NCHUNKS = _EHALF // _CHUNK
_GROUPS = _CHUNK // 16
_HGROUPS = _N // 16


def _sc_energy(x, h, J, ei, ej):
    mesh = plsc.VectorSubcoreMesh(core_axis_name="c", subcore_axis_name="s")

    @functools.partial(
        pl.kernel,
        out_type=jax.ShapeDtypeStruct((_NW, 16), jnp.float32),
        mesh=mesh,
        scratch_types=[
            pltpu.VMEM((_N,), jnp.float32),
            pltpu.VMEM((_N,), jnp.float32),
            pltpu.VMEM((_CHUNK,), jnp.int32),
            pltpu.VMEM((_CHUNK,), jnp.int32),
            pltpu.VMEM((_CHUNK,), jnp.float32),
            pltpu.VMEM((16,), jnp.float32),
        ],
    )
    def body(x_hbm, h_hbm, j_hbm, ei_hbm, ej_hbm, out_hbm, xb, hv, ii, jj, jw, ov):
        c = lax.axis_index("c")
        s = lax.axis_index("s")
        wid = s * _NC + c
        pltpu.sync_copy(x_hbm.at[s], xb)
        ebase = c * _EHALF

        def chunk(ci, acc):
            off = ebase + ci * _CHUNK
            pltpu.sync_copy(ei_hbm.at[pl.ds(off, _CHUNK)], ii)
            pltpu.sync_copy(ej_hbm.at[pl.ds(off, _CHUNK)], jj)
            pltpu.sync_copy(j_hbm.at[pl.ds(off, _CHUNK)], jw)

            def grp(k, a):
                base = k * 16
                iv = ii[pl.ds(base, 16)]
                jv = jj[pl.ds(base, 16)]
                w = jw[pl.ds(base, 16)]
                si = plsc.load_gather(xb, [iv])
                sj = plsc.load_gather(xb, [jv])
                return a + si * sj * w

            return pl.loop(0, _GROUPS, init_carry=acc)(grp)

        acc = pl.loop(0, _NCHUNKS, init_carry=jnp.zeros((16,), jnp.float32))(chunk)
        ov[...] = acc

        @pl.when(c == 0)
        def _():
            pltpu.sync_copy(h_hbm, hv)

            def hgrp(k, a):
                base = k * 16
                return a + hv[pl.ds(base, 16)] * xb[pl.ds(base, 16)]

            hacc = pl.loop(0, _HGROUPS, init_carry=jnp.zeros((16,), jnp.float32))(hgrp)
            ov[...] = ov[...] + hacc

        pltpu.sync_copy(ov, out_hbm.at[wid])

    return body(x, h, J, ei, ej)


def kernel(x, h, J, edge_idx_i, edge_idx_j):
    ei = edge_idx_i.astype(jnp.int32)
    ej = edge_idx_j.astype(jnp.int32)
    parts = _sc_energy(x, h, J, ei, ej)
    return parts.reshape(_B, _NC, 16).sum(axis=(1, 2))


# speedup vs baseline: 10.5304x; 10.5304x over previous
import functools

import jax
import jax.numpy as jnp
from jax import lax
from jax.experimental import pallas as pl
from jax.experimental.pallas import tpu as pltpu
from jax.experimental.pallas import tpu_sc as plsc

_B = 16
_N = 50000
_E = 1600000
_NC = 2
_NS = 16
_NW = _NC * _NS
_EHALF = _E // _NC
_CHUNK = 4000
_NCHUNKS = _EHALF // _CHUNK
_GROUPS = _CHUNK // 16
_HGROUPS = _N // 16


def _sc_energy(x, h, J, ei, ej):
    mesh = plsc.VectorSubcoreMesh(core_axis_name="c", subcore_axis_name="s")

    @functools.partial(
        pl.kernel,
        out_type=jax.ShapeDtypeStruct((_NW, 16), jnp.float32),
        mesh=mesh,
        compiler_params=pltpu.CompilerParams(needs_layout_passes=False),
        scratch_types=[
            pltpu.VMEM((_N,), jnp.float32),
            pltpu.VMEM((_N,), jnp.float32),
            pltpu.VMEM((_CHUNK,), jnp.int32),
            pltpu.VMEM((_CHUNK,), jnp.int32),
            pltpu.VMEM((_CHUNK,), jnp.float32),
            pltpu.VMEM((16,), jnp.float32),
        ],
    )
    def body(x_hbm, h_hbm, j_hbm, ei_hbm, ej_hbm, out_hbm, xb, hv, ii, jj, jw, ov):
        c = lax.axis_index("c")
        s = lax.axis_index("s")
        wid = s * _NC + c
        pltpu.sync_copy(x_hbm.at[s], xb)
        ebase = c * _EHALF

        def chunk(ci, acc):
            off = ebase + ci * _CHUNK
            pltpu.sync_copy(ei_hbm.at[pl.ds(off, _CHUNK)], ii)
            pltpu.sync_copy(ej_hbm.at[pl.ds(off, _CHUNK)], jj)
            pltpu.sync_copy(j_hbm.at[pl.ds(off, _CHUNK)], jw)

            def grp(k, a):
                base = k * 16
                iv = ii[pl.ds(base, 16)]
                jv = jj[pl.ds(base, 16)]
                w = jw[pl.ds(base, 16)]
                si = plsc.load_gather(xb, [iv])
                sj = plsc.load_gather(xb, [jv])
                return a + si * sj * w

            return pl.loop(0, _GROUPS, init_carry=acc)(grp)

        acc = pl.loop(0, _NCHUNKS, init_carry=jnp.zeros((16,), jnp.float32))(chunk)
        ov[...] = acc

        @pl.when(c == 0)
        def _():
            pltpu.sync_copy(h_hbm, hv)

            def hgrp(k, a):
                base = k * 16
                return a + hv[pl.ds(base, 16)] * xb[pl.ds(base, 16)]

            hacc = pl.loop(0, _HGROUPS, init_carry=jnp.zeros((16,), jnp.float32))(hgrp)
            ov[...] = ov[...] + hacc

        pltpu.sync_copy(ov, out_hbm.at[wid])

    return body(x, h, J, ei, ej)


def kernel(x, h, J, edge_idx_i, edge_idx_j):
    ei = edge_idx_i.astype(jnp.int32)
    ej = edge_idx_j.astype(jnp.int32)
    parts = _sc_energy(x, h, J, ei, ej)
    return parts.reshape(_B, _NC, 16).sum(axis=(1, 2))


# speedup vs baseline: 29.8652x; 2.8361x over previous
import functools

import jax
import jax.numpy as jnp
from jax import lax
from jax.experimental import pallas as pl
from jax.experimental.pallas import tpu as pltpu
from jax.experimental.pallas import tpu_sc as plsc

_B = 16
_N = 50000
_E = 1600000
_NC = 2
_NS = 16
_NW = _NC * _NS
_EHALF = _E // _NC
_CHUNK = 4000
_NCHUNKS = _EHALF // _CHUNK
_GROUPS = _CHUNK // 16
_HGROUPS = _N // 16


def _sc_energy(x, h, J, ei, ej):
    mesh = plsc.VectorSubcoreMesh(core_axis_name="c", subcore_axis_name="s")

    @functools.partial(
        pl.kernel,
        out_type=jax.ShapeDtypeStruct((_NW, 16), jnp.float32),
        mesh=mesh,
        compiler_params=pltpu.CompilerParams(needs_layout_passes=False),
        scratch_types=[
            pltpu.VMEM((_N,), jnp.float32),
            pltpu.VMEM((_N,), jnp.float32),
            pltpu.VMEM((_CHUNK,), jnp.int32),
            pltpu.VMEM((_CHUNK,), jnp.int32),
            pltpu.VMEM((_CHUNK,), jnp.float32),
            pltpu.VMEM((_CHUNK,), jnp.int32),
            pltpu.VMEM((_CHUNK,), jnp.int32),
            pltpu.VMEM((_CHUNK,), jnp.float32),
            pltpu.VMEM((16,), jnp.float32),
            pltpu.SemaphoreType.DMA,
            pltpu.SemaphoreType.DMA,
            pltpu.SemaphoreType.DMA,
        ],
    )
    def body(
        x_hbm, h_hbm, j_hbm, ei_hbm, ej_hbm, out_hbm,
        xb, hv, ii0, jj0, jw0, ii1, jj1, jw1, ov, sem0, sem1, hsem,
    ):
        c = lax.axis_index("c")
        s = lax.axis_index("s")
        wid = s * _NC + c
        pltpu.async_copy(h_hbm, hv, hsem)
        pltpu.sync_copy(x_hbm.at[s], xb)
        ebase = c * _EHALF
        bufs = ((ii0, jj0, jw0, sem0), (ii1, jj1, jw1, sem1))

        def start(buf, ci):
            bii, bjj, bjw, sem = buf
            off = ebase + ci * _CHUNK
            pltpu.async_copy(ei_hbm.at[pl.ds(off, _CHUNK)], bii, sem)
            pltpu.async_copy(ej_hbm.at[pl.ds(off, _CHUNK)], bjj, sem)
            pltpu.async_copy(j_hbm.at[pl.ds(off, _CHUNK)], bjw, sem)

        def wait(buf):
            bii, bjj, bjw, sem = buf
            pltpu.make_async_copy(ei_hbm.at[pl.ds(0, _CHUNK)], bii, sem).wait()
            pltpu.make_async_copy(ej_hbm.at[pl.ds(0, _CHUNK)], bjj, sem).wait()
            pltpu.make_async_copy(j_hbm.at[pl.ds(0, _CHUNK)], bjw, sem).wait()

        start(bufs[0], 0)

        def outer(ci, acc):
            for p in range(2):
                buf = bufs[p]
                cur = ci + p
                wait(buf)

                @pl.when(cur + 1 < _NCHUNKS)
                def _():
                    start(bufs[1 - p], cur + 1)

                bii, bjj, bjw, _sem = buf

                def grp(k, a):
                    base = k * 16
                    iv = bii[pl.ds(base, 16)]
                    jv = bjj[pl.ds(base, 16)]
                    w = bjw[pl.ds(base, 16)]
                    si = plsc.load_gather(xb, [iv])
                    sj = plsc.load_gather(xb, [jv])
                    return a + si * sj * w

                acc = pl.loop(0, _GROUPS, init_carry=acc, unroll=10)(grp)
            return acc

        acc = pl.loop(0, _NCHUNKS, step=2, init_carry=jnp.zeros((16,), jnp.float32))(outer)
        ov[...] = acc

        pltpu.make_async_copy(h_hbm, hv, hsem).wait()

        @pl.when(c == 0)
        def _():
            def hgrp(k, a):
                base = k * 16
                return a + hv[pl.ds(base, 16)] * xb[pl.ds(base, 16)]

            hacc = pl.loop(0, _HGROUPS, init_carry=jnp.zeros((16,), jnp.float32), unroll=5)(hgrp)
            ov[...] = ov[...] + hacc

        pltpu.sync_copy(ov, out_hbm.at[wid])

    return body(x, h, J, ei, ej)


def kernel(x, h, J, edge_idx_i, edge_idx_j):
    ei = edge_idx_i.astype(jnp.int32)
    ej = edge_idx_j.astype(jnp.int32)
    parts = _sc_energy(x, h, J, ei, ej)
    return parts.reshape(_B, _NC, 16).sum(axis=(1, 2))


# speedup vs baseline: 51.2070x; 1.7146x over previous
import functools

import jax
import jax.numpy as jnp
from jax import lax
from jax.experimental import pallas as pl
from jax.experimental.pallas import tpu as pltpu
from jax.experimental.pallas import tpu_sc as plsc

_B = 16
_N = 50000
_E = 1600000
_NC = 2
_NS = 16
_NW = _NC * _NS
_NQ = 4
_ESHARD = _E // _NQ
_CHUNK = 4000
_NCHUNKS = _ESHARD // _CHUNK
_GROUPS = _CHUNK // 16


def _sc_energy(x, J, ei, ej):
    mesh = plsc.VectorSubcoreMesh(core_axis_name="c", subcore_axis_name="s")

    @functools.partial(
        pl.kernel,
        out_type=jax.ShapeDtypeStruct((_NW, 2, 16), jnp.float32),
        mesh=mesh,
        compiler_params=pltpu.CompilerParams(needs_layout_passes=False),
        scratch_types=[
            pltpu.VMEM((_N,), jnp.float32),
            pltpu.VMEM((_N,), jnp.float32),
            pltpu.VMEM((_CHUNK,), jnp.int32),
            pltpu.VMEM((_CHUNK,), jnp.int32),
            pltpu.VMEM((_CHUNK,), jnp.float32),
            pltpu.VMEM((_CHUNK,), jnp.int32),
            pltpu.VMEM((_CHUNK,), jnp.int32),
            pltpu.VMEM((_CHUNK,), jnp.float32),
            pltpu.VMEM((2, 16), jnp.float32),
            pltpu.SemaphoreType.DMA,
            pltpu.SemaphoreType.DMA,
        ],
    )
    def body(
        x_hbm, j_hbm, ei_hbm, ej_hbm, out_hbm,
        xb0, xb1, ii0, jj0, jw0, ii1, jj1, jw1, ov, sem0, sem1,
    ):
        c = lax.axis_index("c")
        s = lax.axis_index("s")
        rp = s % 8
        q = (s // 8) * 2 + c
        wid = rp * _NQ + q
        pltpu.sync_copy(x_hbm.at[2 * rp], xb0)
        pltpu.sync_copy(x_hbm.at[2 * rp + 1], xb1)
        ebase = q * _ESHARD
        bufs = ((ii0, jj0, jw0, sem0), (ii1, jj1, jw1, sem1))

        def start(buf, ci):
            bii, bjj, bjw, sem = buf
            off = ebase + ci * _CHUNK
            pltpu.async_copy(ei_hbm.at[pl.ds(off, _CHUNK)], bii, sem)
            pltpu.async_copy(ej_hbm.at[pl.ds(off, _CHUNK)], bjj, sem)
            pltpu.async_copy(j_hbm.at[pl.ds(off, _CHUNK)], bjw, sem)

        def wait(buf):
            bii, bjj, bjw, sem = buf
            pltpu.make_async_copy(ei_hbm.at[pl.ds(0, _CHUNK)], bii, sem).wait()
            pltpu.make_async_copy(ej_hbm.at[pl.ds(0, _CHUNK)], bjj, sem).wait()
            pltpu.make_async_copy(j_hbm.at[pl.ds(0, _CHUNK)], bjw, sem).wait()

        start(bufs[0], 0)

        def outer(ci, acc):
            a0, a1 = acc
            for p in range(2):
                buf = bufs[p]
                cur = ci + p
                wait(buf)

                @pl.when(cur + 1 < _NCHUNKS)
                def _():
                    start(bufs[1 - p], cur + 1)

                bii, bjj, bjw, _sem = buf

                def grp(k, a):
                    g0, g1 = a
                    base = k * 16
                    iv = bii[pl.ds(base, 16)]
                    jv = bjj[pl.ds(base, 16)]
                    w = bjw[pl.ds(base, 16)]
                    g0 = g0 + plsc.load_gather(xb0, [iv]) * plsc.load_gather(xb0, [jv]) * w
                    g1 = g1 + plsc.load_gather(xb1, [iv]) * plsc.load_gather(xb1, [jv]) * w
                    return (g0, g1)

                a0, a1 = pl.loop(0, _GROUPS, init_carry=(a0, a1), unroll=10)(grp)
            return (a0, a1)

        z = jnp.zeros((16,), jnp.float32)
        a0, a1 = pl.loop(0, _NCHUNKS, step=2, init_carry=(z, z))(outer)
        ov[0] = a0
        ov[1] = a1
        pltpu.sync_copy(ov, out_hbm.at[wid])

    return body(x, J, ei, ej)


def _tc_hx(x, h):
    def body(x_ref, h_ref, o_ref):
        o_ref[...] = jnp.sum(x_ref[...] * h_ref[...], axis=1, keepdims=True)

    return pl.pallas_call(
        body,
        out_shape=jax.ShapeDtypeStruct((_B, 1), jnp.float32),
    )(x, h.reshape(1, _N))


def kernel(x, h, J, edge_idx_i, edge_idx_j):
    ei = edge_idx_i.astype(jnp.int32)
    ej = edge_idx_j.astype(jnp.int32)
    parts = _sc_energy(x, J, ei, ej)
    hx = _tc_hx(x, h)
    return parts.reshape(8, _NQ, 2, 16).sum(axis=(1, 3)).reshape(_B) + hx[:, 0]


# speedup vs baseline: 59.8361x; 1.1685x over previous
import functools

import jax
import jax.numpy as jnp
import numpy as np
from jax import lax
from jax.experimental import pallas as pl
from jax.experimental.pallas import tpu as pltpu
from jax.experimental.pallas import tpu_sc as plsc

_B = 16
_N = 50000
_E = 1600000
_NQ = 8
_NV = 4
_NW = 32
_ESHARD = _E // _NQ
_CHUNK = 4000
_NCHUNKS = _ESHARD // _CHUNK
_GROUPS = _CHUNK // 16
_HIMASK = np.int32(-65536)
_LOMASK = np.int32(0xFFFF)


def _sc_energy(xp, J, eij):
    mesh = plsc.VectorSubcoreMesh(core_axis_name="c", subcore_axis_name="s")

    @functools.partial(
        pl.kernel,
        out_type=jax.ShapeDtypeStruct((_NW, 4, 16), jnp.float32),
        mesh=mesh,
        compiler_params=pltpu.CompilerParams(needs_layout_passes=False),
        scratch_types=[
            pltpu.VMEM((_N,), jnp.int32),
            pltpu.VMEM((_N,), jnp.int32),
            pltpu.VMEM((_CHUNK,), jnp.int32),
            pltpu.VMEM((_CHUNK,), jnp.float32),
            pltpu.VMEM((_CHUNK,), jnp.int32),
            pltpu.VMEM((_CHUNK,), jnp.float32),
            pltpu.VMEM((4, 16), jnp.float32),
            pltpu.SemaphoreType.DMA,
            pltpu.SemaphoreType.DMA,
        ],
    )
    def body(
        xp_hbm, j_hbm, eij_hbm, out_hbm,
        pk0, pk1, ee0, jw0, ee1, jw1, ov, sem0, sem1,
    ):
        c = lax.axis_index("c")
        s = lax.axis_index("s")
        v = s % _NV
        q = (s // _NV) * 2 + c
        wid = v * _NQ + q
        pltpu.sync_copy(xp_hbm.at[2 * v], pk0)
        pltpu.sync_copy(xp_hbm.at[2 * v + 1], pk1)
        ebase = q * _ESHARD
        bufs = ((ee0, jw0, sem0), (ee1, jw1, sem1))

        def start(buf, ci):
            bee, bjw, sem = buf
            off = ebase + ci * _CHUNK
            pltpu.async_copy(eij_hbm.at[pl.ds(off, _CHUNK)], bee, sem)
            pltpu.async_copy(j_hbm.at[pl.ds(off, _CHUNK)], bjw, sem)

        def wait(buf):
            bee, bjw, sem = buf
            pltpu.make_async_copy(eij_hbm.at[pl.ds(0, _CHUNK)], bee, sem).wait()
            pltpu.make_async_copy(j_hbm.at[pl.ds(0, _CHUNK)], bjw, sem).wait()

        start(bufs[0], 0)

        def outer(ci, acc):
            accs = acc
            for p in range(2):
                buf = bufs[p]
                cur = ci + p
                wait(buf)

                @pl.when(cur + 1 < _NCHUNKS)
                def _():
                    start(bufs[1 - p], cur + 1)

                bee, bjw, _sem = buf

                def grp(k, a):
                    a0, a1, a2, a3 = a
                    base = k * 16
                    pe = bee[pl.ds(base, 16)]
                    w = bjw[pl.ds(base, 16)]
                    iv = lax.shift_right_logical(pe, 16)
                    jv = lax.bitwise_and(pe, _LOMASK)
                    gi0 = plsc.load_gather(pk0, [iv])
                    gj0 = plsc.load_gather(pk0, [jv])
                    gi1 = plsc.load_gather(pk1, [iv])
                    gj1 = plsc.load_gather(pk1, [jv])

                    def hi(g):
                        return plsc.bitcast(lax.bitwise_and(g, _HIMASK), jnp.float32)

                    def lo(g):
                        return plsc.bitcast(lax.shift_left(g, 16), jnp.float32)

                    a0 = a0 + hi(gi0) * hi(gj0) * w
                    a1 = a1 + lo(gi0) * lo(gj0) * w
                    a2 = a2 + hi(gi1) * hi(gj1) * w
                    a3 = a3 + lo(gi1) * lo(gj1) * w
                    return (a0, a1, a2, a3)

                accs = pl.loop(0, _GROUPS, init_carry=accs, unroll=10)(grp)
            return accs

        z = jnp.zeros((16,), jnp.float32)
        a0, a1, a2, a3 = pl.loop(0, _NCHUNKS, step=2, init_carry=(z, z, z, z))(outer)
        ov[0] = a0
        ov[1] = a1
        ov[2] = a2
        ov[3] = a3
        pltpu.sync_copy(ov, out_hbm.at[wid])

    return body(xp, J, eij)


def _tc_hx(x, h):
    def body(x_ref, h_ref, o_ref):
        o_ref[...] = jnp.sum(x_ref[...] * h_ref[...], axis=1, keepdims=True)

    return pl.pallas_call(
        body,
        out_shape=jax.ShapeDtypeStruct((_B, 1), jnp.float32),
    )(x, h.reshape(1, _N))


def kernel(x, h, J, edge_idx_i, edge_idx_j):
    u = lax.bitcast_convert_type(x.astype(jnp.bfloat16), jnp.uint16).astype(jnp.uint32)
    xp = lax.bitcast_convert_type((u[0::2] << 16) | u[1::2], jnp.int32)
    eij = lax.bitcast_convert_type(
        (edge_idx_i.astype(jnp.uint32) << 16) | edge_idx_j.astype(jnp.uint32),
        jnp.int32,
    )
    parts = _sc_energy(xp, J, eij)
    hx = _tc_hx(x, h)
    return parts.reshape(_NV, _NQ, 4, 16).sum(axis=(1, 3)).reshape(_B) + hx[:, 0]


# speedup vs baseline: 70.5970x; 1.1798x over previous
import functools

import jax
import jax.numpy as jnp
import numpy as np
from jax import lax
from jax.experimental import pallas as pl
from jax.experimental.pallas import tpu as pltpu
from jax.experimental.pallas import tpu_sc as plsc

_B = 16
_N = 50000
_E = 1600000
_NQ = 8
_NV = 4
_NW = 32
_ESHARD = _E // _NQ
_CHUNK = 4000
_NCHUNKS = _ESHARD // _CHUNK
_GROUPS = _CHUNK // 16
_HIMASK = np.int32(-65536)
_LOMASK = np.int32(0xFFFF)


def _sc_energy(xp, J, ei, ej):
    mesh = plsc.VectorSubcoreMesh(core_axis_name="c", subcore_axis_name="s")

    @functools.partial(
        pl.kernel,
        out_type=jax.ShapeDtypeStruct((_NW, 4, 16), jnp.float32),
        mesh=mesh,
        compiler_params=pltpu.CompilerParams(needs_layout_passes=False),
        scratch_types=[
            pltpu.VMEM((_N,), jnp.int32),
            pltpu.VMEM((_N,), jnp.int32),
            pltpu.VMEM((_CHUNK,), jnp.int32),
            pltpu.VMEM((_CHUNK,), jnp.int32),
            pltpu.VMEM((_CHUNK,), jnp.float32),
            pltpu.VMEM((_CHUNK,), jnp.int32),
            pltpu.VMEM((_CHUNK,), jnp.int32),
            pltpu.VMEM((_CHUNK,), jnp.float32),
            pltpu.VMEM((4, 16), jnp.float32),
            pltpu.SemaphoreType.DMA,
            pltpu.SemaphoreType.DMA,
        ],
    )
    def body(
        xp_hbm, j_hbm, ei_hbm, ej_hbm, out_hbm,
        pk0, pk1, ii0, jj0, jw0, ii1, jj1, jw1, ov, sem0, sem1,
    ):
        c = lax.axis_index("c")
        s = lax.axis_index("s")
        v = s % _NV
        q = (s // _NV) * 2 + c
        wid = v * _NQ + q
        pltpu.sync_copy(xp_hbm.at[2 * v], pk0)
        pltpu.sync_copy(xp_hbm.at[2 * v + 1], pk1)
        ebase = q * _ESHARD
        bufs = ((ii0, jj0, jw0, sem0), (ii1, jj1, jw1, sem1))

        def start(buf, ci):
            bii, bjj, bjw, sem = buf
            off = ebase + ci * _CHUNK
            pltpu.async_copy(ei_hbm.at[pl.ds(off, _CHUNK)], bii, sem)
            pltpu.async_copy(ej_hbm.at[pl.ds(off, _CHUNK)], bjj, sem)
            pltpu.async_copy(j_hbm.at[pl.ds(off, _CHUNK)], bjw, sem)

        def wait(buf):
            bii, bjj, bjw, sem = buf
            pltpu.make_async_copy(ei_hbm.at[pl.ds(0, _CHUNK)], bii, sem).wait()
            pltpu.make_async_copy(ej_hbm.at[pl.ds(0, _CHUNK)], bjj, sem).wait()
            pltpu.make_async_copy(j_hbm.at[pl.ds(0, _CHUNK)], bjw, sem).wait()

        start(bufs[0], 0)

        def outer(ci, acc):
            accs = acc
            for p in range(2):
                buf = bufs[p]
                cur = ci + p
                wait(buf)

                @pl.when(cur + 1 < _NCHUNKS)
                def _():
                    start(bufs[1 - p], cur + 1)

                bii, bjj, bjw, _sem = buf

                def grp(k, a):
                    a0, a1, a2, a3 = a
                    base = k * 16
                    iv = bii[pl.ds(base, 16)]
                    jv = bjj[pl.ds(base, 16)]
                    w = bjw[pl.ds(base, 16)]
                    gi0 = plsc.load_gather(pk0, [iv])
                    gj0 = plsc.load_gather(pk0, [jv])
                    gi1 = plsc.load_gather(pk1, [iv])
                    gj1 = plsc.load_gather(pk1, [jv])

                    def hi(g):
                        return plsc.bitcast(lax.bitwise_and(g, _HIMASK), jnp.float32)

                    def lo(g):
                        return plsc.bitcast(lax.shift_left(g, 16), jnp.float32)

                    a0 = a0 + hi(gi0) * hi(gj0) * w
                    a1 = a1 + lo(gi0) * lo(gj0) * w
                    a2 = a2 + hi(gi1) * hi(gj1) * w
                    a3 = a3 + lo(gi1) * lo(gj1) * w
                    return (a0, a1, a2, a3)

                accs = pl.loop(0, _GROUPS, init_carry=accs, unroll=10)(grp)
            return accs

        z = jnp.zeros((16,), jnp.float32)
        a0, a1, a2, a3 = pl.loop(0, _NCHUNKS, step=2, init_carry=(z, z, z, z))(outer)
        ov[0] = a0
        ov[1] = a1
        ov[2] = a2
        ov[3] = a3
        pltpu.sync_copy(ov, out_hbm.at[wid])

    return body(xp, J, ei, ej)


def _tc_hx(x, h):
    def body(x_ref, h_ref, o_ref):
        o_ref[...] = jnp.sum(x_ref[...] * h_ref[...], axis=1, keepdims=True)

    return pl.pallas_call(
        body,
        out_shape=jax.ShapeDtypeStruct((_B, 1), jnp.float32),
    )(x, h.reshape(1, _N))


def kernel(x, h, J, edge_idx_i, edge_idx_j):
    u = lax.bitcast_convert_type(x.astype(jnp.bfloat16), jnp.uint16).astype(jnp.uint32)
    xp = lax.bitcast_convert_type((u[0::2] << 16) | u[1::2], jnp.int32)
    ei = edge_idx_i.astype(jnp.int32)
    ej = edge_idx_j.astype(jnp.int32)
    parts = _sc_energy(xp, J, ei, ej)
    hx = _tc_hx(x, h)
    return parts.reshape(_NV, _NQ, 4, 16).sum(axis=(1, 3)).reshape(_B) + hx[:, 0]


# speedup vs baseline: 77.0606x; 1.0916x over previous
import functools

import jax
import jax.numpy as jnp
import numpy as np
from jax import lax
from jax.experimental import pallas as pl
from jax.experimental.pallas import tpu as pltpu
from jax.experimental.pallas import tpu_sc as plsc

_B = 16
_N = 50000
_E = 1600000
_NQ = 8
_NV = 4
_NW = 32
_ESHARD = _E // _NQ
_CHUNK = 4000
_NCHUNKS = _ESHARD // _CHUNK
_GROUPS = _CHUNK // 16
_HIMASK = np.int32(-65536)
_LOMASK = np.int32(0xFFFF)


def _sc_energy(xp, J, ei, ej):
    mesh = plsc.VectorSubcoreMesh(core_axis_name="c", subcore_axis_name="s")

    @functools.partial(
        pl.kernel,
        out_type=jax.ShapeDtypeStruct((_NW, 4, 16), jnp.float32),
        mesh=mesh,
        compiler_params=pltpu.CompilerParams(needs_layout_passes=False),
        scratch_types=[
            pltpu.VMEM((_N,), jnp.int32),
            pltpu.VMEM((_N,), jnp.int32),
            pltpu.VMEM((_CHUNK,), jnp.int32),
            pltpu.VMEM((_CHUNK,), jnp.int32),
            pltpu.VMEM((_CHUNK,), jnp.float32),
            pltpu.VMEM((_CHUNK,), jnp.int32),
            pltpu.VMEM((_CHUNK,), jnp.int32),
            pltpu.VMEM((_CHUNK,), jnp.float32),
            pltpu.VMEM((4, 16), jnp.float32),
            pltpu.SemaphoreType.DMA,
            pltpu.SemaphoreType.DMA,
        ],
    )
    def body(
        xp_hbm, j_hbm, ei_hbm, ej_hbm, out_hbm,
        pk0, pk1, ii0, jj0, jw0, ii1, jj1, jw1, ov, sem0, sem1,
    ):
        c = lax.axis_index("c")
        s = lax.axis_index("s")
        v = s % _NV
        q = (s // _NV) * 2 + c
        wid = v * _NQ + q
        pltpu.sync_copy(xp_hbm.at[2 * v], pk0)
        pltpu.sync_copy(xp_hbm.at[2 * v + 1], pk1)
        ebase = q * _ESHARD
        bufs = ((ii0, jj0, jw0, sem0), (ii1, jj1, jw1, sem1))

        def start(buf, ci):
            bii, bjj, bjw, sem = buf
            off = ebase + ci * _CHUNK
            pltpu.async_copy(ei_hbm.at[pl.ds(off, _CHUNK)], bii, sem)
            pltpu.async_copy(ej_hbm.at[pl.ds(off, _CHUNK)], bjj, sem)
            pltpu.async_copy(j_hbm.at[pl.ds(off, _CHUNK)], bjw, sem)

        def wait(buf):
            bii, bjj, bjw, sem = buf
            pltpu.make_async_copy(ei_hbm.at[pl.ds(0, _CHUNK)], bii, sem).wait()
            pltpu.make_async_copy(ej_hbm.at[pl.ds(0, _CHUNK)], bjj, sem).wait()
            pltpu.make_async_copy(j_hbm.at[pl.ds(0, _CHUNK)], bjw, sem).wait()

        start(bufs[0], 0)

        def outer(ci, acc):
            accs = acc
            for p in range(2):
                buf = bufs[p]
                cur = ci + p
                wait(buf)

                @pl.when(cur + 1 < _NCHUNKS)
                def _():
                    start(bufs[1 - p], cur + 1)

                bii, bjj, bjw, _sem = buf

                def grp(k, a):
                    a0, a1, a2, a3 = a
                    base = k * 16
                    iv = bii[pl.ds(base, 16)]
                    jv = bjj[pl.ds(base, 16)]
                    w = bjw[pl.ds(base, 16)]
                    gi0 = plsc.load_gather(pk0, [iv])
                    gj0 = plsc.load_gather(pk0, [jv])
                    gi1 = plsc.load_gather(pk1, [iv])
                    gj1 = plsc.load_gather(pk1, [jv])

                    def hi(g):
                        return plsc.bitcast(lax.bitwise_and(g, _HIMASK), jnp.float32)

                    def lo(g):
                        return plsc.bitcast(lax.shift_left(g, 16), jnp.float32)

                    a0 = a0 + hi(gi0) * hi(gj0) * w
                    a1 = a1 + lo(gi0) * lo(gj0) * w
                    a2 = a2 + hi(gi1) * hi(gj1) * w
                    a3 = a3 + lo(gi1) * lo(gj1) * w
                    return (a0, a1, a2, a3)

                accs = pl.loop(0, _GROUPS, init_carry=accs, unroll=10)(grp)
            return accs

        z = jnp.zeros((16,), jnp.float32)
        a0, a1, a2, a3 = pl.loop(0, _NCHUNKS, step=2, init_carry=(z, z, z, z))(outer)
        ov[0] = a0
        ov[1] = a1
        ov[2] = a2
        ov[3] = a3
        pltpu.sync_copy(ov, out_hbm.at[wid])

    return body(xp, J, ei, ej)


def _tc_prep(x, h):
    def body(x_ref, h_ref, xp_ref, hx_ref):
        xf = x_ref[...]
        u = lax.bitcast_convert_type(xf, jnp.uint32)

        def rn(v):
            return (v + jnp.uint32(0x7FFF) + ((v >> 16) & jnp.uint32(1))) & jnp.uint32(
                0xFFFF0000
            )

        packed = rn(u[0:8]) | (rn(u[8:16]) >> 16)
        xp_ref[...] = lax.bitcast_convert_type(packed, jnp.int32)
        hx_ref[...] = jnp.sum(xf * h_ref[...], axis=1, keepdims=True)

    return pl.pallas_call(
        body,
        out_shape=[
            jax.ShapeDtypeStruct((8, _N), jnp.int32),
            jax.ShapeDtypeStruct((_B, 1), jnp.float32),
        ],
    )(x, h.reshape(1, _N))


_ROW_ORDER = np.argsort(
    np.array([[2 * v, 2 * v + 8, 2 * v + 1, 2 * v + 9] for v in range(_NV)]).reshape(-1)
)


def kernel(x, h, J, edge_idx_i, edge_idx_j):
    xp, hx = _tc_prep(x, h)
    ei = edge_idx_i.astype(jnp.int32)
    ej = edge_idx_j.astype(jnp.int32)
    parts = _sc_energy(xp, J, ei, ej)
    r = parts.reshape(_NV, _NQ, 4, 16).sum(axis=(1, 3)).reshape(_B)
    return r[_ROW_ORDER] + hx[:, 0]
